# Initial kernel scaffold; baseline (speedup 1.0000x reference)
import jax
import jax.numpy as jnp
from jax import lax
from jax.experimental import pallas as pl
from jax.experimental.pallas import tpu as pltpu
from jax.experimental.pallas import tpu_sc as plsc

_N = 10000
_D = 128
_E = 320000
_ITERS = 3

_NC, _NS = 2, 16
_NW = _NC * _NS
_CHUNK = 128
_NCHUNK = 80
_E_PAD = _NW * _NCHUNK * _CHUNK
_N_PAD = 10240
_ROWS_PER_TILE = _N_PAD // _NS

_mesh = plsc.VectorSubcoreMesh(
    core_axis_name="c", subcore_axis_name="s",
    num_cores=_NC, num_subcores=_NS)


def _agg_body(cur_hbm, src_hbm, dst_hbm, zrows_hbm, out_hbm,
              src_v, dst_v, rows0, rows1, acc_sh, sem0, sem1):
    cid = lax.axis_index("c")
    sid = lax.axis_index("s")
    wid = sid * _NC + cid

    pltpu.sync_copy(zrows_hbm,
                    acc_sh.at[pl.ds(sid * _ROWS_PER_TILE, _ROWS_PER_TILE)])
    base = wid * _NCHUNK
    pltpu.sync_copy(src_hbm.at[pl.ds(base, _NCHUNK)], src_v)
    pltpu.sync_copy(dst_hbm.at[pl.ds(base, _NCHUNK)], dst_v)
    plsc.subcore_barrier()

    def gather(c, buf, sem):
        pltpu.async_copy(cur_hbm.at[src_v.at[c]], buf, sem)

    def gwait(buf, sem):
        pltpu.make_async_copy(cur_hbm.at[src_v.at[0]], buf, sem).wait()

    def scat(c, buf):
        pltpu.sync_copy(buf, acc_sh.at[dst_v.at[c]], add=True)

    gather(0, rows0, sem0)
    gather(1, rows1, sem1)

    def body(c, carry):
        gwait(rows0, sem0)
        scat(2 * c, rows0)
        gather(2 * c + 2, rows0, sem0)
        gwait(rows1, sem1)
        scat(2 * c + 1, rows1)
        gather(2 * c + 3, rows1, sem1)
        return carry

    lax.fori_loop(0, _NCHUNK // 2 - 1, body, 0)
    gwait(rows0, sem0)
    scat(_NCHUNK - 2, rows0)
    gwait(rows1, sem1)
    scat(_NCHUNK - 1, rows1)

    plsc.subcore_barrier()
    pltpu.sync_copy(acc_sh.at[pl.ds(sid * _ROWS_PER_TILE, _ROWS_PER_TILE)],
                    out_hbm.at[cid, pl.ds(sid * _ROWS_PER_TILE, _ROWS_PER_TILE)])


_agg = pl.kernel(
    _agg_body,
    out_type=jax.ShapeDtypeStruct((_NC, _N_PAD, _D), jnp.float32),
    mesh=_mesh,
    scratch_types=[
        pltpu.VMEM((_NCHUNK, _CHUNK), jnp.int32),
        pltpu.VMEM((_NCHUNK, _CHUNK), jnp.int32),
        pltpu.VMEM((_CHUNK, _D), jnp.float32),
        pltpu.VMEM((_CHUNK, _D), jnp.float32),
        pltpu.VMEM_SHARED((_N_PAD, _D), jnp.float32),
        pltpu.SemaphoreType.DMA,
        pltpu.SemaphoreType.DMA,
    ],
)


def _cnt_body(dst_hbm, zvec_hbm, ones_hbm, out_hbm,
              dst_v, ones_v, cnt_sh):
    cid = lax.axis_index("c")
    sid = lax.axis_index("s")
    wid = sid * _NC + cid

    pltpu.sync_copy(zvec_hbm,
                    cnt_sh.at[pl.ds(sid * _ROWS_PER_TILE, _ROWS_PER_TILE)])
    pltpu.sync_copy(ones_hbm, ones_v)
    base = wid * _NCHUNK
    pltpu.sync_copy(dst_hbm.at[pl.ds(base, _NCHUNK)], dst_v)
    plsc.subcore_barrier()

    def body(c, carry):
        pltpu.sync_copy(ones_v, cnt_sh.at[dst_v.at[c]], add=True)
        return carry

    lax.fori_loop(0, _NCHUNK, body, 0)

    plsc.subcore_barrier()
    pltpu.sync_copy(cnt_sh.at[pl.ds(sid * _ROWS_PER_TILE, _ROWS_PER_TILE)],
                    out_hbm.at[cid, pl.ds(sid * _ROWS_PER_TILE, _ROWS_PER_TILE)])


_count = pl.kernel(
    _cnt_body,
    out_type=jax.ShapeDtypeStruct((_NC, _N_PAD), jnp.float32),
    mesh=_mesh,
    scratch_types=[
        pltpu.VMEM((_NCHUNK, _CHUNK), jnp.int32),
        pltpu.VMEM((_CHUNK,), jnp.float32),
        pltpu.VMEM_SHARED((_N_PAD,), jnp.float32),
    ],
)


_BN = 2000


def _fuse_body(p_ref, cur_ref, w_ref, b_ref, cnt_ref, out_ref):
    rows = p_ref[0] + p_ref[1]
    m = jnp.dot(rows, w_ref[...], preferred_element_type=jnp.float32)
    c = cnt_ref[0] + cnt_ref[1]
    denom = jnp.where(c > 0.0, c, 1.0)
    out_ref[...] = cur_ref[...] + jnp.maximum(m / denom + b_ref[...], 0.0)


def _fuse(p, cur, W, b2, cnt3):
    return pl.pallas_call(
        _fuse_body,
        grid=(_N // _BN,),
        in_specs=[
            pl.BlockSpec((_NC, _BN, _D), lambda g: (0, g, 0)),
            pl.BlockSpec((_BN, _D), lambda g: (g, 0)),
            pl.BlockSpec((_D, _D), lambda g: (0, 0)),
            pl.BlockSpec((1, _D), lambda g: (0, 0)),
            pl.BlockSpec((_NC, _BN, 1), lambda g: (0, g, 0)),
        ],
        out_specs=pl.BlockSpec((_BN, _D), lambda g: (g, 0)),
        out_shape=jax.ShapeDtypeStruct((_N, _D), jnp.float32),
    )(p, cur, W, b2, cnt3)


def kernel(x, edge_index, W, b):
    src = edge_index[0]
    dst = edge_index[1]
    pad = _E_PAD - _E
    src_p = jnp.concatenate(
        [src, jnp.zeros((pad,), jnp.int32)]).reshape(_NW * _NCHUNK, _CHUNK)
    dst_p = jnp.concatenate(
        [dst, _N + (jnp.arange(pad, dtype=jnp.int32) % 8)]
    ).reshape(_NW * _NCHUNK, _CHUNK)
    zrows = jnp.zeros((_ROWS_PER_TILE, _D), jnp.float32)
    zvec = jnp.zeros((_ROWS_PER_TILE,), jnp.float32)
    ones = jnp.ones((_CHUNK,), jnp.float32)
    b2 = b.reshape(1, _D)

    cnt = _count(dst_p, zvec, ones)
    cnt3 = cnt.reshape(_NC, _N_PAD, 1)
    cur = x
    for _ in range(_ITERS):
        p = _agg(cur, src_p, dst_p, zrows)
        cur = _fuse(p, cur, W, b2, cnt3)
    return cur


# speedup vs baseline: 3.1744x; 3.1744x over previous
import jax
import jax.numpy as jnp
from jax import lax
from jax.experimental import pallas as pl
from jax.experimental.pallas import tpu as pltpu
from jax.experimental.pallas import tpu_sc as plsc

_N = 10000
_D = 128
_E = 320000
_ITERS = 3

_NC, _NS = 2, 16
_NW = _NC * _NS
_CHUNK = 128
_NCHUNK = 80
_STAGE = 16
_E_PAD = _NW * _NCHUNK * _CHUNK
_N_PAD = 10240
_ROWS_PER_TILE = _N_PAD // _NS

_mesh = plsc.VectorSubcoreMesh(
    core_axis_name="c", subcore_axis_name="s",
    num_cores=_NC, num_subcores=_NS)


def _agg_body(cur_hbm, src_hbm, dst_hbm, zrows_hbm, out_hbm,
              src_v, dst_v, rows0, rows1, acc_sh, sem0, sem1):
    cid = lax.axis_index("c")
    sid = lax.axis_index("s")
    wid = sid * _NC + cid

    pltpu.sync_copy(zrows_hbm,
                    acc_sh.at[pl.ds(sid * _ROWS_PER_TILE, _ROWS_PER_TILE)])
    plsc.subcore_barrier()

    base = wid * _NCHUNK

    def gather(c, buf, sem):
        pltpu.async_copy(cur_hbm.at[src_v.at[c]], buf, sem)

    def gwait(buf, sem):
        pltpu.make_async_copy(cur_hbm.at[src_v.at[0]], buf, sem).wait()

    def scat(c, buf):
        pltpu.sync_copy(buf, acc_sh.at[dst_v.at[c]], add=True)

    def stage_body(st, carry):
        pltpu.sync_copy(src_hbm.at[pl.ds(base + st * _STAGE, _STAGE)], src_v)
        pltpu.sync_copy(dst_hbm.at[pl.ds(base + st * _STAGE, _STAGE)], dst_v)
        gather(0, rows0, sem0)
        gather(1, rows1, sem1)

        def body(c, carry2):
            gwait(rows0, sem0)
            scat(2 * c, rows0)
            gather(2 * c + 2, rows0, sem0)
            gwait(rows1, sem1)
            scat(2 * c + 1, rows1)
            gather(2 * c + 3, rows1, sem1)
            return carry2

        lax.fori_loop(0, _STAGE // 2 - 1, body, 0)
        gwait(rows0, sem0)
        scat(_STAGE - 2, rows0)
        gwait(rows1, sem1)
        scat(_STAGE - 1, rows1)
        return carry

    lax.fori_loop(0, _NCHUNK // _STAGE, stage_body, 0)

    plsc.subcore_barrier()
    pltpu.sync_copy(acc_sh.at[pl.ds(sid * _ROWS_PER_TILE, _ROWS_PER_TILE)],
                    out_hbm.at[cid, pl.ds(sid * _ROWS_PER_TILE, _ROWS_PER_TILE)])


_agg = pl.kernel(
    _agg_body,
    out_type=jax.ShapeDtypeStruct((_NC, _N_PAD, _D), jnp.float32),
    mesh=_mesh,
    scratch_types=[
        pltpu.VMEM((_STAGE, _CHUNK), jnp.int32),
        pltpu.VMEM((_STAGE, _CHUNK), jnp.int32),
        pltpu.VMEM((_CHUNK, _D), jnp.float32),
        pltpu.VMEM((_CHUNK, _D), jnp.float32),
        pltpu.VMEM_SHARED((_N_PAD, _D), jnp.float32),
        pltpu.SemaphoreType.DMA,
        pltpu.SemaphoreType.DMA,
    ],
)


def _cnt_body(dst_hbm, zvec_hbm, ones_hbm, out_hbm,
              dst_v, ones_v, cnt_sh):
    cid = lax.axis_index("c")
    sid = lax.axis_index("s")
    wid = sid * _NC + cid

    pltpu.sync_copy(zvec_hbm,
                    cnt_sh.at[pl.ds(sid * _ROWS_PER_TILE, _ROWS_PER_TILE)])
    pltpu.sync_copy(ones_hbm, ones_v)
    base = wid * _NCHUNK
    pltpu.sync_copy(dst_hbm.at[pl.ds(base, _NCHUNK)], dst_v)
    plsc.subcore_barrier()

    def body(c, carry):
        pltpu.sync_copy(ones_v, cnt_sh.at[dst_v.at[c]], add=True)
        return carry

    lax.fori_loop(0, _NCHUNK, body, 0)

    plsc.subcore_barrier()
    pltpu.sync_copy(cnt_sh.at[pl.ds(sid * _ROWS_PER_TILE, _ROWS_PER_TILE)],
                    out_hbm.at[cid, pl.ds(sid * _ROWS_PER_TILE, _ROWS_PER_TILE)])


_count = pl.kernel(
    _cnt_body,
    out_type=jax.ShapeDtypeStruct((_NC, _N_PAD), jnp.float32),
    mesh=_mesh,
    scratch_types=[
        pltpu.VMEM((_NCHUNK, _CHUNK), jnp.int32),
        pltpu.VMEM((_CHUNK,), jnp.float32),
        pltpu.VMEM_SHARED((_N_PAD,), jnp.float32),
    ],
)


_BN = 2000


def _fuse_body(p_ref, cur_ref, w_ref, b_ref, cnt_ref, out_ref):
    rows = p_ref[0] + p_ref[1]
    m = jnp.dot(rows, w_ref[...], preferred_element_type=jnp.float32)
    c = cnt_ref[0] + cnt_ref[1]
    denom = jnp.where(c > 0.0, c, 1.0)
    out_ref[...] = cur_ref[...] + jnp.maximum(m / denom + b_ref[...], 0.0)


def _fuse(p, cur, W, b2, cnt3):
    return pl.pallas_call(
        _fuse_body,
        grid=(_N // _BN,),
        in_specs=[
            pl.BlockSpec((_NC, _BN, _D), lambda g: (0, g, 0)),
            pl.BlockSpec((_BN, _D), lambda g: (g, 0)),
            pl.BlockSpec((_D, _D), lambda g: (0, 0)),
            pl.BlockSpec((1, _D), lambda g: (0, 0)),
            pl.BlockSpec((_NC, _BN, 1), lambda g: (0, g, 0)),
        ],
        out_specs=pl.BlockSpec((_BN, _D), lambda g: (g, 0)),
        out_shape=jax.ShapeDtypeStruct((_N, _D), jnp.float32),
    )(p, cur, W, b2, cnt3)


def kernel(x, edge_index, W, b):
    src = edge_index[0]
    dst = edge_index[1]
    pad = _E_PAD - _E
    src_p = jnp.concatenate(
        [src, jnp.zeros((pad,), jnp.int32)]).reshape(_NW * _NCHUNK, _CHUNK)
    dst_p = jnp.concatenate(
        [dst, _N + (jnp.arange(pad, dtype=jnp.int32) % 8)]
    ).reshape(_NW * _NCHUNK, _CHUNK)
    zrows = jnp.zeros((_ROWS_PER_TILE, _D), jnp.float32)
    zvec = jnp.zeros((_ROWS_PER_TILE,), jnp.float32)
    ones = jnp.ones((_CHUNK,), jnp.float32)
    b2 = b.reshape(1, _D)

    cnt = _count(dst_p, zvec, ones)
    cnt3 = cnt.reshape(_NC, _N_PAD, 1)
    cur = x
    for _ in range(_ITERS):
        p = _agg(cur, src_p, dst_p, zrows)
        cur = _fuse(p, cur, W, b2, cnt3)
    return cur


# speedup vs baseline: 3.2501x; 1.0239x over previous
import jax
import jax.numpy as jnp
from jax import lax
from jax.experimental import pallas as pl
from jax.experimental.pallas import tpu as pltpu
from jax.experimental.pallas import tpu_sc as plsc

_N = 10000
_D = 128
_E = 320000
_ITERS = 3

_NC, _NS = 2, 16
_NW = _NC * _NS
_CHUNK = 128
_NCHUNK = 80
_STAGE = 16
_E_PAD = _NW * _NCHUNK * _CHUNK
_N_PAD = 10240
_ROWS_PER_TILE = _N_PAD // _NS

_mesh = plsc.VectorSubcoreMesh(
    core_axis_name="c", subcore_axis_name="s",
    num_cores=_NC, num_subcores=_NS)


def _agg_body(cur_hbm, src_hbm, dst_hbm, zrows_hbm, out_hbm,
              src_v, dst_v, rows0, rows1, acc_sh, sem0, sem1):
    cid = lax.axis_index("c")
    sid = lax.axis_index("s")
    wid = sid * _NC + cid

    pltpu.sync_copy(zrows_hbm,
                    acc_sh.at[pl.ds(sid * _ROWS_PER_TILE, _ROWS_PER_TILE)])
    plsc.subcore_barrier()

    base = wid * _NCHUNK

    def gather(c, buf, sem):
        pltpu.async_copy(cur_hbm.at[src_v.at[c]], buf, sem)

    def gwait(buf, sem):
        pltpu.make_async_copy(cur_hbm.at[src_v.at[0]], buf, sem).wait()

    def scat(c, buf):
        pltpu.sync_copy(buf, acc_sh.at[dst_v.at[c]], add=True)

    def stage_body(st, carry):
        pltpu.sync_copy(src_hbm.at[pl.ds(base + st * _STAGE, _STAGE)], src_v)
        pltpu.sync_copy(dst_hbm.at[pl.ds(base + st * _STAGE, _STAGE)], dst_v)
        gather(0, rows0, sem0)
        gather(1, rows1, sem1)

        def body(c, carry2):
            gwait(rows0, sem0)
            scat(2 * c, rows0)
            gather(2 * c + 2, rows0, sem0)
            gwait(rows1, sem1)
            scat(2 * c + 1, rows1)
            gather(2 * c + 3, rows1, sem1)
            return carry2

        lax.fori_loop(0, _STAGE // 2 - 1, body, 0)
        gwait(rows0, sem0)
        scat(_STAGE - 2, rows0)
        gwait(rows1, sem1)
        scat(_STAGE - 1, rows1)
        return carry

    lax.fori_loop(0, _NCHUNK // _STAGE, stage_body, 0)

    plsc.subcore_barrier()
    pltpu.sync_copy(acc_sh.at[pl.ds(sid * _ROWS_PER_TILE, _ROWS_PER_TILE)],
                    out_hbm.at[cid, pl.ds(sid * _ROWS_PER_TILE, _ROWS_PER_TILE)])


_agg = pl.kernel(
    _agg_body,
    out_type=jax.ShapeDtypeStruct((_NC, _N_PAD, _D), jnp.float32),
    mesh=_mesh,
    scratch_types=[
        pltpu.VMEM((_STAGE, _CHUNK), jnp.int32),
        pltpu.VMEM((_STAGE, _CHUNK), jnp.int32),
        pltpu.VMEM((_CHUNK, _D), jnp.float32),
        pltpu.VMEM((_CHUNK, _D), jnp.float32),
        pltpu.VMEM_SHARED((_N_PAD, _D), jnp.float32),
        pltpu.SemaphoreType.DMA,
        pltpu.SemaphoreType.DMA,
    ],
)


def _cnt_body(dst_hbm, zvec_hbm, ones_hbm, out_hbm,
              dst_v, ones_v, cnt_sh):
    cid = lax.axis_index("c")
    sid = lax.axis_index("s")
    wid = sid * _NC + cid

    pltpu.sync_copy(zvec_hbm,
                    cnt_sh.at[pl.ds(sid * _ROWS_PER_TILE, _ROWS_PER_TILE)])
    pltpu.sync_copy(ones_hbm, ones_v)
    base = wid * _NCHUNK
    pltpu.sync_copy(dst_hbm.at[pl.ds(base, _NCHUNK)], dst_v)
    plsc.subcore_barrier()

    def body(c, carry):
        pltpu.sync_copy(ones_v, cnt_sh.at[dst_v.at[c]], add=True)
        return carry

    lax.fori_loop(0, _NCHUNK, body, 0)

    plsc.subcore_barrier()
    pltpu.sync_copy(cnt_sh.at[pl.ds(sid * _ROWS_PER_TILE, _ROWS_PER_TILE)],
                    out_hbm.at[cid, pl.ds(sid * _ROWS_PER_TILE, _ROWS_PER_TILE)])


_count = pl.kernel(
    _cnt_body,
    out_type=jax.ShapeDtypeStruct((_NC, _N_PAD), jnp.float32),
    mesh=_mesh,
    scratch_types=[
        pltpu.VMEM((_NCHUNK, _CHUNK), jnp.int32),
        pltpu.VMEM((_CHUNK,), jnp.float32),
        pltpu.VMEM_SHARED((_N_PAD,), jnp.float32),
    ],
)


_BN = 2000


def _fuse_body(p_ref, cur_ref, w_ref, b_ref, cnt_ref, out_ref):
    rows = p_ref[0] + p_ref[1]
    m = jnp.dot(rows, w_ref[...], preferred_element_type=jnp.float32)
    c = cnt_ref[0] + cnt_ref[1]
    denom = jnp.where(c > 0.0, c, 1.0)
    out_ref[...] = cur_ref[...] + jnp.maximum(m / denom + b_ref[...], 0.0)


def _fuse(p, cur, W, b2, cnt3):
    return pl.pallas_call(
        _fuse_body,
        grid=(_N // _BN,),
        in_specs=[
            pl.BlockSpec((_NC, _BN, _D), lambda g: (0, g, 0)),
            pl.BlockSpec((_BN, _D), lambda g: (g, 0)),
            pl.BlockSpec((_D, _D), lambda g: (0, 0)),
            pl.BlockSpec((1, _D), lambda g: (0, 0)),
            pl.BlockSpec((_NC, _BN, 1), lambda g: (0, g, 0)),
        ],
        out_specs=pl.BlockSpec((_BN, _D), lambda g: (g, 0)),
        out_shape=jax.ShapeDtypeStruct((_N, _D), jnp.float32),
    )(p, cur, W, b2, cnt3)


def kernel(x, edge_index, W, b):
    src = edge_index[0]
    dst = edge_index[1]
    pad = _E_PAD - _E
    src_p = jnp.concatenate(
        [src, jnp.zeros((pad,), jnp.int32)]).reshape(_NW * _NCHUNK, _CHUNK)
    dst_p = jnp.concatenate(
        [dst, _N + (jnp.arange(pad, dtype=jnp.int32) % (_N_PAD - _N))]
    ).reshape(_NW * _NCHUNK, _CHUNK)
    zrows = jnp.zeros((_ROWS_PER_TILE, _D), jnp.float32)
    zvec = jnp.zeros((_ROWS_PER_TILE,), jnp.float32)
    ones = jnp.ones((_CHUNK,), jnp.float32)
    b2 = b.reshape(1, _D)

    cnt = _count(dst_p, zvec, ones)
    cnt3 = cnt.reshape(_NC, _N_PAD, 1)
    cur = x
    for _ in range(_ITERS):
        p = _agg(cur, src_p, dst_p, zrows)
        cur = _fuse(p, cur, W, b2, cnt3)
    return cur


# speedup vs baseline: 11.6469x; 3.5835x over previous
import jax
import jax.numpy as jnp
from jax import lax
from jax.experimental import pallas as pl
from jax.experimental.pallas import tpu as pltpu
from jax.experimental.pallas import tpu_sc as plsc

_N = 10000
_D = 128
_E = 320000
_ITERS = 3

_NC, _NS = 2, 16
_NW = _NC * _NS
_CHUNK = 128
_NCHUNK = 80
_STAGE = 16
_NBUF = 2
_E_PAD = _NW * _NCHUNK * _CHUNK
_N_PAD = 10240
_ROWS_PER_TILE = _N_PAD // _NS
_N_PAD_CNT = 10240
_CNT_PER_TILE = _N_PAD_CNT // _NS

_mesh = plsc.VectorSubcoreMesh(
    core_axis_name="c", subcore_axis_name="s",
    num_cores=_NC, num_subcores=_NS)


def _agg_body(cur_hbm, ei_hbm, zrows_hbm, out_hbm,
              idx_v, rows, gsems, acc_sh):
    cid = lax.axis_index("c")
    sid = lax.axis_index("s")
    wid = sid * _NC + cid

    pltpu.sync_copy(zrows_hbm,
                    acc_sh.at[pl.ds(sid * _ROWS_PER_TILE, _ROWS_PER_TILE)])
    plsc.subcore_barrier()

    base = wid * _NCHUNK

    def gather(j, b):
        pltpu.async_copy(cur_hbm.at[idx_v.at[2 * j]], rows[b], gsems[b])

    def gwait(b):
        pltpu.make_async_copy(cur_hbm.at[idx_v.at[0]], rows[b], gsems[b]).wait()

    def scat(j, b):
        pltpu.sync_copy(rows[b], acc_sh.at[idx_v.at[2 * j + 1]], add=True)

    def stage_body(st, carry):
        pltpu.sync_copy(
            ei_hbm.at[pl.ds(2 * (base + st * _STAGE), 2 * _STAGE)], idx_v)
        gather(0, 0)
        gather(1, 1)
        for j in range(_STAGE):
            b = j % _NBUF
            gwait(b)
            scat(j, b)
            if j + 2 < _STAGE:
                gather(j + 2, b)
        return carry

    lax.fori_loop(0, _NCHUNK // _STAGE, stage_body, 0)

    plsc.subcore_barrier()
    pltpu.sync_copy(acc_sh.at[pl.ds(sid * _ROWS_PER_TILE, _ROWS_PER_TILE)],
                    out_hbm.at[cid, pl.ds(sid * _ROWS_PER_TILE, _ROWS_PER_TILE)])


_agg = pl.kernel(
    _agg_body,
    out_type=jax.ShapeDtypeStruct((_NC, _N_PAD, _D), jnp.float32),
    mesh=_mesh,
    scratch_types=[
        pltpu.VMEM((2 * _STAGE, _CHUNK), jnp.int32),
        [pltpu.VMEM((_CHUNK, _D), jnp.float32) for _ in range(_NBUF)],
        [pltpu.SemaphoreType.DMA for _ in range(_NBUF)],
        pltpu.VMEM_SHARED((_N_PAD, _D), jnp.float32),
    ],
)


def _cnt_body(dst_hbm, zvec_hbm, ones_hbm, out_hbm,
              dst_v, ones_v, cnt_sh):
    cid = lax.axis_index("c")
    sid = lax.axis_index("s")
    wid = sid * _NC + cid

    pltpu.sync_copy(zvec_hbm,
                    cnt_sh.at[pl.ds(sid * _CNT_PER_TILE, _CNT_PER_TILE)])
    pltpu.sync_copy(ones_hbm, ones_v)
    base = wid * _NCHUNK
    pltpu.sync_copy(dst_hbm.at[pl.ds(base, _NCHUNK)], dst_v)
    plsc.subcore_barrier()

    def body(c, carry):
        pltpu.sync_copy(ones_v, cnt_sh.at[dst_v.at[c]], add=True)
        return carry

    lax.fori_loop(0, _NCHUNK, body, 0)

    plsc.subcore_barrier()
    pltpu.sync_copy(cnt_sh.at[pl.ds(sid * _CNT_PER_TILE, _CNT_PER_TILE)],
                    out_hbm.at[cid, pl.ds(sid * _CNT_PER_TILE, _CNT_PER_TILE)])


_count = pl.kernel(
    _cnt_body,
    out_type=jax.ShapeDtypeStruct((_NC, _N_PAD_CNT), jnp.float32),
    mesh=_mesh,
    scratch_types=[
        pltpu.VMEM((_NCHUNK, _CHUNK), jnp.int32),
        pltpu.VMEM((_CHUNK,), jnp.float32),
        pltpu.VMEM_SHARED((_N_PAD_CNT,), jnp.float32),
    ],
)


_BN = 2000


def _fuse_body(p_ref, cur_ref, w_ref, b_ref, cnt_ref, out_ref):
    rows = p_ref[0] + p_ref[1]
    m = jnp.dot(rows, w_ref[...], preferred_element_type=jnp.float32)
    c = cnt_ref[0] + cnt_ref[1]
    denom = jnp.where(c > 0.0, c, 1.0)
    out_ref[...] = cur_ref[...] + jnp.maximum(m / denom + b_ref[...], 0.0)


def _fuse(p, cur, W, b2, cnt3):
    return pl.pallas_call(
        _fuse_body,
        grid=(_N // _BN,),
        in_specs=[
            pl.BlockSpec((_NC, _BN, _D), lambda g: (0, g, 0)),
            pl.BlockSpec((_BN, _D), lambda g: (g, 0)),
            pl.BlockSpec((_D, _D), lambda g: (0, 0)),
            pl.BlockSpec((1, _D), lambda g: (0, 0)),
            pl.BlockSpec((_NC, _BN, 1), lambda g: (0, g, 0)),
        ],
        out_specs=pl.BlockSpec((_BN, _D), lambda g: (g, 0)),
        out_shape=jax.ShapeDtypeStruct((_N, _D), jnp.float32),
    )(p, cur, W, b2, cnt3)


def kernel(x, edge_index, W, b):
    src = edge_index[0]
    dst = edge_index[1]
    pad = _E_PAD - _E
    src_p = jnp.concatenate(
        [src, jnp.arange(pad, dtype=jnp.int32) % _N]
    ).reshape(_NW * _NCHUNK, _CHUNK)
    dst_p = jnp.concatenate(
        [dst, _N + (jnp.arange(pad, dtype=jnp.int32) % (_N_PAD - _N))]
    ).reshape(_NW * _NCHUNK, _CHUNK)
    ei = jnp.stack([src_p, dst_p], axis=1).reshape(2 * _NW * _NCHUNK, _CHUNK)
    zrows = jnp.zeros((_ROWS_PER_TILE, _D), jnp.float32)
    zvec = jnp.zeros((_CNT_PER_TILE,), jnp.float32)
    ones = jnp.ones((_CHUNK,), jnp.float32)
    b2 = b.reshape(1, _D)

    cnt = _count(dst_p, zvec, ones)
    cnt3 = cnt.reshape(_NC, _N_PAD_CNT, 1)
    cur = x
    for _ in range(_ITERS):
        p = _agg(cur, ei, zrows)
        cur = _fuse(p, cur, W, b2, cnt3)
    return cur
